# baseline (speedup 1.0000x reference)
import functools

import jax
import jax.numpy as jnp
import numpy as np
from jax import lax
from jax.experimental import pallas as pl
from jax.experimental.pallas import tpu as pltpu
from jax.experimental.pallas import tpu_sc as plsc

_NC = 2
_NS = 16
_NW = _NC * _NS
_B = 16384
_D2 = 32
_BPW = _B // _NW
_CH = 128
_NCH = _BPW // _CH

_HALF_LN_2PI = 0.9189385332046727


def _select_rows(rows, idxs, c, g, outbuf):
    rel = g * 16 + lax.iota(jnp.int32, 16)
    off32 = (idxs[c, pl.ds(g * 16, 16)] & 3) * 32
    pbase = c * 4096 + rel * 32
    for d in range(_D2):
        vals = plsc.load_gather(rows, [rel, off32 + d])
        p = pbase + d
        plsc.store_scatter(outbuf, [p >> 7, p & 127], vals)


def _gather_body(uq_hbm, mq_hbm, uo_hbm, mo_hbm, ut4, mt4, bu_hbm, bm_hbm,
                 u_out, m_out, bu_out, bm_out,
                 uqidx, mqidx, uoidx, moidx, urows, mrows, ubuf, mbuf,
                 buv, bmv, sem, bias_sem):
    wid = lax.axis_index("s") * _NC + lax.axis_index("c")
    rbase = wid * _NCH
    obase = wid * (_BPW * _D2 // 128)

    pltpu.sync_copy(uq_hbm.at[pl.ds(rbase, _NCH)], uqidx)
    pltpu.sync_copy(mq_hbm.at[pl.ds(rbase, _NCH)], mqidx)
    pltpu.sync_copy(uo_hbm.at[pl.ds(rbase, _NCH)], uoidx)
    pltpu.sync_copy(mo_hbm.at[pl.ds(rbase, _NCH)], moidx)

    bias_cps = []
    for j in range(_NCH):
        bias_cps.append(pltpu.async_copy(bu_hbm.at[uqidx.at[j]], buv.at[j],
                                         bias_sem))
        bias_cps.append(pltpu.async_copy(bm_hbm.at[mqidx.at[j]], bmv.at[j],
                                         bias_sem))

    def fire(c, slot):
        return (pltpu.async_copy(ut4.at[uoidx.at[c]], urows.at[slot], sem),
                pltpu.async_copy(mt4.at[moidx.at[c]], mrows.at[slot], sem))

    pend = fire(0, 0)
    for c in range(_NCH):
        for cp in pend:
            cp.wait()
        slot = c & 1
        if c + 1 < _NCH:
            pend = fire(c + 1, (c + 1) & 1)

        def body(g, _):
            _select_rows(urows.at[slot], uqidx, c, g, ubuf)
            _select_rows(mrows.at[slot], mqidx, c, g, mbuf)
            return 0

        lax.fori_loop(0, _CH // 16, body, 0)

    for cp in bias_cps:
        cp.wait()
    pltpu.sync_copy(ubuf, u_out.at[pl.ds(obase, _BPW * _D2 // 128)])
    pltpu.sync_copy(mbuf, m_out.at[pl.ds(obase, _BPW * _D2 // 128)])
    pltpu.sync_copy(buv, bu_out.at[pl.ds(rbase, _NCH)])
    pltpu.sync_copy(bmv, bm_out.at[pl.ds(rbase, _NCH)])


def _sc_gather(users_q2d, movies_q2d, users_o2d, movies_o2d, ut4, mt4,
               Bu, Bm):
    mesh = plsc.VectorSubcoreMesh(core_axis_name="c", subcore_axis_name="s")
    f32 = jnp.float32
    i32 = jnp.int32
    return pl.kernel(
        _gather_body,
        out_type=(
            jax.ShapeDtypeStruct((_B * _D2 // 128, 128), f32),
            jax.ShapeDtypeStruct((_B * _D2 // 128, 128), f32),
            jax.ShapeDtypeStruct((_NW * _NCH, _CH), f32),
            jax.ShapeDtypeStruct((_NW * _NCH, _CH), f32),
        ),
        mesh=mesh,
        scratch_types=[
            pltpu.VMEM((_NCH, _CH), i32),
            pltpu.VMEM((_NCH, _CH), i32),
            pltpu.VMEM((_NCH, _CH), i32),
            pltpu.VMEM((_NCH, _CH), i32),
            pltpu.VMEM((2, _CH, 128), f32),
            pltpu.VMEM((2, _CH, 128), f32),
            pltpu.VMEM((128, 128), f32),
            pltpu.VMEM((128, 128), f32),
            pltpu.VMEM((_NCH, _CH), f32),
            pltpu.VMEM((_NCH, _CH), f32),
            pltpu.SemaphoreType.DMA,
            pltpu.SemaphoreType.DMA,
        ],
        compiler_params=pltpu.CompilerParams(needs_layout_passes=False),
    )(users_q2d, movies_q2d, users_o2d, movies_o2d, ut4, mt4, Bu, Bm)


def _lgamma_big(x, lx, r):
    return (x - 0.5) * lx - x + _HALF_LN_2PI + r * (1.0 / 12.0 - r * r * (1.0 / 360.0))


def _digamma_big(lx, r):
    r2 = r * r
    return lx - r * 0.5 - r2 * (1.0 / 12.0) + r2 * r2 * (1.0 / 120.0)


def _atan(t):
    a = jnp.abs(t)
    inv = 1.0 / jnp.maximum(a, 1e-30)
    z = jnp.minimum(a, inv)
    z2 = z * z
    p = z * (0.9998660 + z2 * (-0.3302995 + z2 * (0.1801410
            + z2 * (-0.0851330 + z2 * 0.0208351))))
    r = jnp.where(a > 1.0, (jnp.pi / 2.0) - p, p)
    return jnp.sign(t) * r


def _math_body(u_ref, m_ref, bu_ref, bm_ref, s_ref, o_ref):
    u = u_ref[...]
    m = m_ref[...]
    u = jnp.where(u != u, 0.05, u)
    m = jnp.where(m != m, 0.05, m)
    u = jnp.clip(u + 1.0, 1.0, 100.0)
    m = jnp.clip(m + 1.0, 1.0, 100.0)

    au = u
    am = m
    bu = jnp.concatenate([u[:, 16:], u[:, :16]], axis=1)
    bm = jnp.concatenate([m[:, 16:], m[:, :16]], axis=1)

    c1m = 0.5 * (bu + bm)
    c0m = 0.5 * (au + am)
    sq = c1m + c0m
    spu = au + bu
    spm = am + bm

    def lg_psi(x):
        lx = jnp.log(x)
        r = 1.0 / x
        return _lgamma_big(x, lx, r), _digamma_big(lx, r)

    def lg(x):
        return _lgamma_big(x, jnp.log(x), 1.0 / x)

    lg_au, ps_au = lg_psi(au)
    lg_bu, ps_bu = lg_psi(bu)
    lg_am, ps_am = lg_psi(am)
    lg_bm, ps_bm = lg_psi(bm)
    lg_spu, ps_spu = lg_psi(spu)
    lg_spm, ps_spm = lg_psi(spm)

    t1x2 = 2.0 * (lg(c1m) + lg(c0m) - lg(sq))
    kl12 = (t1x2
            - lg_au - lg_bu + lg_spu
            - lg_am - lg_bm + lg_spm
            + (au - c1m) * ps_au + (bu - c0m) * ps_bu + (sq - spu) * ps_spu
            + (am - c1m) * ps_am + (bm - c0m) * ps_bm + (sq - spm) * ps_spm)

    t = jnp.abs((2.0 / jnp.pi) * _atan(0.5 * kl12))
    dist = jnp.dot(t, s_ref[...], preferred_element_type=jnp.float32)
    o_ref[...] = bu_ref[...] + bm_ref[...] - dist


def _tc_math(u128, m128, bu4, bm4, sel):
    nrows = _B * _D2 // 128
    blk = 512
    grid = (nrows // blk,)
    return pl.pallas_call(
        _math_body,
        grid=grid,
        in_specs=[
            pl.BlockSpec((blk, 128), lambda i: (i, 0)),
            pl.BlockSpec((blk, 128), lambda i: (i, 0)),
            pl.BlockSpec((blk, 4), lambda i: (i, 0)),
            pl.BlockSpec((blk, 4), lambda i: (i, 0)),
            pl.BlockSpec((128, 4), lambda i: (0, 0)),
        ],
        out_specs=pl.BlockSpec((blk, 4), lambda i: (i, 0)),
        out_shape=jax.ShapeDtypeStruct((nrows, 4), jnp.float32),
    )(u128, m128, bu4, bm4, sel)


@functools.partial(jax.jit, static_argnums=())
def kernel(x, Bu, Bm, u_table, m_table):
    users = x[:, 0]
    movies = x[:, 1]
    users2d = users.reshape(_NW * _NCH, _CH)
    movies2d = movies.reshape(_NW * _NCH, _CH)
    users_q2d = (users >> 2).reshape(_NW * _NCH, _CH)
    movies_q2d = (movies >> 2).reshape(_NW * _NCH, _CH)
    ut4 = u_table.reshape(-1, 128)
    mt4 = m_table.reshape(-1, 128)

    u128, m128, bu_g, bm_g = _sc_gather(users2d, movies2d, users_q2d,
                                        movies_q2d, ut4, mt4, Bu, Bm)

    bu4 = bu_g.reshape(_B // 4, 4)
    bm4 = bm_g.reshape(_B // 4, 4)

    lane = np.arange(128)
    sel = ((lane % 32 < 16)[:, None]
           & ((lane // 32)[:, None] == np.arange(4)[None, :])
           ).astype(np.float32)
    out4 = _tc_math(u128, m128, bu4, bm4, jnp.asarray(sel))
    return out4.reshape(_B)


# speedup vs baseline: 1.0479x; 1.0479x over previous
import functools

import jax
import jax.numpy as jnp
import numpy as np
from jax import lax
from jax.experimental import pallas as pl
from jax.experimental.pallas import tpu as pltpu
from jax.experimental.pallas import tpu_sc as plsc

_NC = 2
_NS = 16
_NW = _NC * _NS
_B = 16384
_D2 = 32
_BPW = _B // _NW
_CH = 128
_NCH = _BPW // _CH

_HALF_LN_2PI = 0.9189385332046727


def _gather_body(idx_hbm, table, bias_hbm, rows_out, bias_out,
                 idx, rows, biasv, sem, bias_sem):
    wid = lax.axis_index("s") * _NC + lax.axis_index("c")
    rbase = wid * _NCH
    base = wid * _BPW

    pltpu.sync_copy(idx_hbm.at[pl.ds(rbase, _NCH)], idx)

    cps = []
    for j in range(_NCH):
        cps.append(pltpu.async_copy(
            table.at[idx.at[j]], rows.at[pl.ds(j * _CH, _CH)], sem))
        cps.append(pltpu.async_copy(bias_hbm.at[idx.at[j]], biasv.at[j],
                                    bias_sem))
    for cp in cps:
        cp.wait()

    pltpu.sync_copy(rows, rows_out.at[pl.ds(base, _BPW)])
    pltpu.sync_copy(biasv, bias_out.at[pl.ds(rbase, _NCH)])


def _sc_gather(idx2d, table, bias):
    mesh = plsc.VectorSubcoreMesh(core_axis_name="c", subcore_axis_name="s")
    f32 = jnp.float32
    return pl.kernel(
        _gather_body,
        out_type=(
            jax.ShapeDtypeStruct((_B, _D2), f32),
            jax.ShapeDtypeStruct((_NW * _NCH, _CH), f32),
        ),
        mesh=mesh,
        scratch_types=[
            pltpu.VMEM((_NCH, _CH), jnp.int32),
            pltpu.VMEM((_BPW, _D2), f32),
            pltpu.VMEM((_NCH, _CH), f32),
            pltpu.SemaphoreType.DMA,
            pltpu.SemaphoreType.DMA,
        ],
        compiler_params=pltpu.CompilerParams(use_tc_tiling_on_sc=False),
    )(idx2d, table, bias)


def _lgamma_big(x, lx, r):
    return (x - 0.5) * lx - x + _HALF_LN_2PI + r * (1.0 / 12.0 - r * r * (1.0 / 360.0))


def _digamma_big(lx, r):
    r2 = r * r
    return lx - r * 0.5 - r2 * (1.0 / 12.0) + r2 * r2 * (1.0 / 120.0)


def _atan(t):
    a = jnp.abs(t)
    inv = 1.0 / jnp.maximum(a, 1e-30)
    z = jnp.minimum(a, inv)
    z2 = z * z
    p = z * (0.9998660 + z2 * (-0.3302995 + z2 * (0.1801410
            + z2 * (-0.0851330 + z2 * 0.0208351))))
    r = jnp.where(a > 1.0, (jnp.pi / 2.0) - p, p)
    return jnp.sign(t) * r


def _math_body(u_ref, m_ref, bu_ref, bm_ref, s_ref, o_ref):
    u = u_ref[...]
    m = m_ref[...]
    u = jnp.where(u != u, 0.05, u)
    m = jnp.where(m != m, 0.05, m)
    u = jnp.clip(u + 1.0, 1.0, 100.0)
    m = jnp.clip(m + 1.0, 1.0, 100.0)

    au = u
    am = m
    bu = jnp.concatenate([u[:, 16:], u[:, :16]], axis=1)
    bm = jnp.concatenate([m[:, 16:], m[:, :16]], axis=1)

    c1m = 0.5 * (bu + bm)
    c0m = 0.5 * (au + am)
    sq = c1m + c0m
    spu = au + bu
    spm = am + bm

    def lg_psi(x):
        lx = jnp.log(x)
        r = 1.0 / x
        return _lgamma_big(x, lx, r), _digamma_big(lx, r)

    def lg(x):
        return _lgamma_big(x, jnp.log(x), 1.0 / x)

    lg_au, ps_au = lg_psi(au)
    lg_bu, ps_bu = lg_psi(bu)
    lg_am, ps_am = lg_psi(am)
    lg_bm, ps_bm = lg_psi(bm)
    lg_spu, ps_spu = lg_psi(spu)
    lg_spm, ps_spm = lg_psi(spm)

    t1x2 = 2.0 * (lg(c1m) + lg(c0m) - lg(sq))
    kl12 = (t1x2
            - lg_au - lg_bu + lg_spu
            - lg_am - lg_bm + lg_spm
            + (au - c1m) * ps_au + (bu - c0m) * ps_bu + (sq - spu) * ps_spu
            + (am - c1m) * ps_am + (bm - c0m) * ps_bm + (sq - spm) * ps_spm)

    t = jnp.abs((2.0 / jnp.pi) * _atan(0.5 * kl12))
    dist = jnp.dot(t, s_ref[...], preferred_element_type=jnp.float32)
    o_ref[...] = bu_ref[...] + bm_ref[...] - dist


def _tc_math(u128, m128, bu4, bm4, sel):
    nrows = _B * _D2 // 128
    blk = 512
    grid = (nrows // blk,)
    return pl.pallas_call(
        _math_body,
        grid=grid,
        in_specs=[
            pl.BlockSpec((blk, 128), lambda i: (i, 0)),
            pl.BlockSpec((blk, 128), lambda i: (i, 0)),
            pl.BlockSpec((blk, 4), lambda i: (i, 0)),
            pl.BlockSpec((blk, 4), lambda i: (i, 0)),
            pl.BlockSpec((128, 4), lambda i: (0, 0)),
        ],
        out_specs=pl.BlockSpec((blk, 4), lambda i: (i, 0)),
        out_shape=jax.ShapeDtypeStruct((nrows, 4), jnp.float32),
    )(u128, m128, bu4, bm4, sel)


@functools.partial(jax.jit, static_argnums=())
def kernel(x, Bu, Bm, u_table, m_table):
    users2d = x[:, 0].reshape(_NW * _NCH, _CH)
    movies2d = x[:, 1].reshape(_NW * _NCH, _CH)

    u_g, bu_g = _sc_gather(users2d, u_table, Bu)
    m_g, bm_g = _sc_gather(movies2d, m_table, Bm)

    u128 = u_g.reshape(_B * _D2 // 128, 128)
    m128 = m_g.reshape(_B * _D2 // 128, 128)
    bu4 = bu_g.reshape(_B // 4, 4)
    bm4 = bm_g.reshape(_B // 4, 4)

    lane = np.arange(128)
    sel = ((lane % 32 < 16)[:, None]
           & ((lane // 32)[:, None] == np.arange(4)[None, :])
           ).astype(np.float32)
    out4 = _tc_math(u128, m128, bu4, bm4, jnp.asarray(sel))
    return out4.reshape(_B)
